# Initial kernel scaffold; baseline (speedup 1.0000x reference)
import jax
import jax.numpy as jnp
from jax.experimental import pallas as pl


def kernel(x_room, x_sensor, edge_index_sr, edge_index_rr, params):
    raise NotImplementedError("write your pallas kernel here")


# speedup vs baseline: 11.1537x; 11.1537x over previous
import functools

import jax
import jax.numpy as jnp
from jax import lax
from jax.experimental import pallas as pl
from jax.experimental.pallas import tpu as pltpu, tpu_sc as plsc

N_NODE = 10000
NPAD = 10240
T_HIS = 12
ST = 32
NPASS = 3
W = 4 * ST
E_TOT = 160000
NC, NS = 2, 16
HALF = NPAD // NC
ACC_ROWS = HALF + 8
CHUNK = 128
NCHUNK = 80
EPT = CHUNK * NCHUNK
E_PAD = EPT * NS
RPT = HALF // NS
BN = 512


def _stage1_call(xT, w1fl, b1, a1T, relT, gselfT):
    cin = xT.shape[2]
    with_self = gselfT is not None
    grid = NPAD // BN

    def body(*refs):
        if with_self:
            x_ref, w_ref, b_ref, a_ref, r_ref, g_ref, m_ref, s_ref = refs
        else:
            x_ref, w_ref, b_ref, a_ref, r_ref, m_ref = refs
        x = x_ref[...]
        w = w_ref[...]
        b = b_ref[...]
        aT = a_ref[...]
        rT = r_ref[...]
        z = jnp.zeros((BN, ST), jnp.float32)
        ms, ss = [], []
        for t in range(10):
            xw = jnp.concatenate([x[:, t, :], x[:, t + 1, :], x[:, t + 2, :]],
                                 axis=-1)
            y = jnp.dot(xw, w, preferred_element_type=jnp.float32) + b
            p, q = y[:, :ST], y[:, ST:]
            xa = jnp.dot(x[:, t + 2, :], aT, preferred_element_type=jnp.float32)
            h = (p + xa) * jax.nn.sigmoid(q)
            ms.append(jnp.dot(h, rT, preferred_element_type=jnp.float32))
            if with_self:
                ss.append(jnp.dot(h, g_ref[...],
                                  preferred_element_type=jnp.float32))
        ms += [z, z]
        ss += [z, z]
        for pp in range(NPASS):
            m_ref[:, pp, :] = jnp.concatenate(ms[4 * pp:4 * pp + 4], axis=-1)
            if with_self:
                s_ref[pp, :, :] = jnp.concatenate(ss[4 * pp:4 * pp + 4],
                                                  axis=-1)

    full = lambda *s: pl.BlockSpec(s, lambda i: (0,) * len(s))
    in_specs = [
        pl.BlockSpec((BN, T_HIS, cin), lambda i: (i, 0, 0)),
        full(3 * cin, 2 * ST),
        full(1, 2 * ST),
        full(cin, ST),
        full(ST, ST),
    ]
    out_specs = [pl.BlockSpec((BN, NPASS, W), lambda i: (i, 0, 0))]
    out_shape = [jax.ShapeDtypeStruct((NPAD, NPASS, W), jnp.float32)]
    args = [xT, w1fl, b1, a1T, relT]
    if with_self:
        in_specs.append(full(ST, ST))
        out_specs.append(pl.BlockSpec((NPASS, BN, W), lambda i: (0, i, 0)))
        out_shape.append(jax.ShapeDtypeStruct((NPASS, NPAD, W), jnp.float32))
        args.append(gselfT)
    out = pl.pallas_call(
        body, grid=(grid,), in_specs=in_specs, out_specs=out_specs,
        out_shape=out_shape)(*args)
    return out if with_self else (out[0], None)


def _sc_aggregate(mm3, ma3, ss, src_sr, dst_sr, src_rr, dst_rr):
    mesh = plsc.VectorSubcoreMesh(core_axis_name="c", subcore_axis_name="s",
                                  num_cores=NC, num_subcores=NS)

    @functools.partial(
        pl.kernel,
        out_type=jax.ShapeDtypeStruct((NPASS, NPAD, W), jnp.float32),
        mesh=mesh,
        scratch_types=[
            pltpu.VMEM((CHUNK,), jnp.int32),
            pltpu.VMEM((CHUNK,), jnp.int32),
            pltpu.VMEM((CHUNK,), jnp.int32),
            pltpu.VMEM((CHUNK,), jnp.int32),
            pltpu.VMEM((CHUNK, W), jnp.float32),
            pltpu.VMEM_SHARED((ACC_ROWS, W), jnp.float32),
            pltpu.SemaphoreType.DMA,
        ],
    )
    def k(mm_hbm, ma_hbm, ss_hbm, ssr_hbm, dsr_hbm, srr_hbm, drr_hbm, out_hbm,
          srcv, dstv, src3v, dlocv, rows, acc, sem):
        cid = lax.axis_index("c")
        sid = lax.axis_index("s")
        base = sid * RPT
        for p in range(NPASS):
            pltpu.sync_copy(ss_hbm.at[p, pl.ds(cid * HALF + base, RPT)],
                            acc.at[pl.ds(base, RPT)])
            plsc.subcore_barrier()
            for tab, s_hbm, d_hbm in ((mm_hbm, ssr_hbm, dsr_hbm),
                                      (ma_hbm, srr_hbm, drr_hbm)):
                ebase = sid * EPT

                def chunk(i, _, tab=tab, s_hbm=s_hbm, d_hbm=d_hbm, p=p):
                    e0 = ebase + i * CHUNK
                    pltpu.sync_copy(s_hbm.at[pl.ds(e0, CHUNK)], srcv)
                    pltpu.sync_copy(d_hbm.at[pl.ds(e0, CHUNK)], dstv)
                    for g in range(CHUNK // 16):
                        sl = pl.ds(g * 16, 16)
                        src3v[sl] = srcv[sl] * 3 + p
                        dl = dstv[sl] - cid * HALF
                        ok = (dl >= 0) & (dl < HALF)
                        dlocv[sl] = jnp.where(ok, dl, HALF)
                    pltpu.async_copy(tab.at[src3v], rows, sem).wait()
                    pltpu.async_copy(rows, acc.at[dlocv], sem, add=True).wait()
                    return 0

                lax.fori_loop(0, NCHUNK, chunk, 0)
            plsc.subcore_barrier()
            pltpu.sync_copy(acc.at[pl.ds(base, RPT)],
                            out_hbm.at[p, pl.ds(cid * HALF + base, RPT)])
            if p + 1 < NPASS:
                plsc.subcore_barrier()

    return k(mm3, ma3, ss, src_sr, dst_sr, src_rr, dst_rr)


def _stage2_call(agg3, w2fl, b2, a2T, lng, lnb, twT, tb, taT, lng2, lnb2,
                 f1T, f1b, f2T, f2b):
    grid = NPAD // BN

    def body(ag_ref, w2_ref, b2_ref, a2_ref, g_ref, bt_ref, tw_ref, tb_ref,
             ta_ref, g2_ref, bt2_ref, f1_ref, f1b_ref, f2_ref, f2b_ref, o_ref):
        ags = []
        for pp in range(NPASS):
            blkp = jnp.maximum(ag_ref[pp], 0.0)
            for j in range(4):
                ags.append(blkp[:, j * ST:(j + 1) * ST])
        w2 = w2_ref[...]
        hs = []
        for t in range(8):
            xw = jnp.concatenate([ags[t], ags[t + 1], ags[t + 2]], axis=-1)
            y = jnp.dot(xw, w2, preferred_element_type=jnp.float32) + b2_ref[...]
            p, q = y[:, :ST], y[:, ST:]
            xa = jnp.dot(ags[t + 2], a2_ref[...],
                         preferred_element_type=jnp.float32)
            h = (p + xa) * jax.nn.sigmoid(q)
            mu = jnp.mean(h, axis=-1, keepdims=True)
            var = jnp.mean((h - mu) ** 2, axis=-1, keepdims=True)
            hs.append((h - mu) * lax.rsqrt(var + 1e-5) * g_ref[...]
                      + bt_ref[...])
        acc = tb_ref[...] + jnp.zeros((BN, 256), jnp.float32)
        for t in range(8):
            acc = acc + jnp.dot(hs[t], tw_ref[t],
                                preferred_element_type=jnp.float32)
        p, q = acc[:, :128], acc[:, 128:]
        xa = jnp.dot(hs[7], ta_ref[...], preferred_element_type=jnp.float32)
        z = (p + xa) * jax.nn.sigmoid(q)
        mu = jnp.mean(z, axis=-1, keepdims=True)
        var = jnp.mean((z - mu) ** 2, axis=-1, keepdims=True)
        z = (z - mu) * lax.rsqrt(var + 1e-5) * g2_ref[...] + bt2_ref[...]
        z = jnp.maximum(
            jnp.dot(z, f1_ref[...], preferred_element_type=jnp.float32)
            + f1b_ref[...], 0.0)
        o_ref[...] = (jnp.dot(z, f2_ref[...], preferred_element_type=jnp.float32)
                      + f2b_ref[...])

    full = lambda *s: pl.BlockSpec(s, lambda i: (0,) * len(s))
    return pl.pallas_call(
        body,
        grid=(grid,),
        in_specs=[
            pl.BlockSpec((NPASS, BN, W), lambda i: (0, i, 0)),
            full(3 * ST, 2 * ST), full(1, 2 * ST), full(ST, ST),
            full(1, ST), full(1, ST),
            full(8, ST, 256), full(1, 256), full(ST, 128),
            full(1, 128), full(1, 128),
            full(128, 128), full(1, 128), full(128, 8), full(1, 8),
        ],
        out_specs=pl.BlockSpec((BN, 8), lambda i: (i, 0)),
        out_shape=jax.ShapeDtypeStruct((NPAD, 8), jnp.float32),
    )(agg3, w2fl, b2, a2T, lng, lnb, twT, tb, taT, lng2, lnb2, f1T, f1b,
      f2T, f2b)


def _node_major(x, npad):
    xt = jnp.transpose(x[0], (2, 1, 0))
    return jnp.pad(xt, ((0, npad - xt.shape[0]), (0, 0), (0, 0)))


def _conv_flat(w):
    return jnp.transpose(w[..., 0], (2, 1, 0)).reshape(-1, w.shape[0])


def _pad_edges(ei):
    src = jnp.pad(ei[0], (0, E_PAD - E_TOT), constant_values=0)
    dst = jnp.pad(ei[1], (0, E_PAD - E_TOT), constant_values=1 << 20)
    return src, dst


def kernel(x_room, x_sensor, edge_index_sr, edge_index_rr, params):
    blk = params["blocks"][0]
    po = params["out"]
    br, bs = blk["room"], blk["sensor"]

    xr = _node_major(x_room, NPAD)
    xs = _node_major(x_sensor, NPAD)

    m_adj, s_self = _stage1_call(
        xr, _conv_flat(br["t1_w"]), br["t1_b"].reshape(1, -1),
        br["t1_align"].T, blk["rel"]["adjacent"].T, br["g_self"].T)
    m_meas, _ = _stage1_call(
        xs, _conv_flat(bs["t1_w"]), bs["t1_b"].reshape(1, -1),
        bs["t1_align"].T, blk["rel"]["measures"].T, None)

    ssr, dsr = _pad_edges(edge_index_sr)
    srr, drr = _pad_edges(edge_index_rr)
    agg3 = _sc_aggregate(
        m_meas.reshape(NPASS * NPAD, W), m_adj.reshape(NPASS * NPAD, W),
        s_self, ssr, dsr, srr, drr)

    f2T = jnp.zeros((128, 8), jnp.float32).at[:, :3].set(po["fc2_w"].T)
    f2b = jnp.zeros((1, 8), jnp.float32).at[:, :3].set(po["fc2_b"][None, :])
    out = _stage2_call(
        agg3,
        _conv_flat(br["t2_w"]), br["t2_b"].reshape(1, -1), br["t2_align"].T,
        br["ln_g"].reshape(1, -1), br["ln_b"].reshape(1, -1),
        jnp.transpose(po["t_w"][..., 0], (2, 1, 0)),
        po["t_b"].reshape(1, -1), po["t_align"].T,
        po["ln_g"].reshape(1, -1), po["ln_b"].reshape(1, -1),
        po["fc1_w"].T, po["fc1_b"].reshape(1, -1), f2T, f2b)

    y = out[:N_NODE, :3]
    return jnp.transpose(y, (1, 0))[None, :, :]


# speedup vs baseline: 12.0332x; 1.0789x over previous
import functools

import jax
import jax.numpy as jnp
from jax import lax
from jax.experimental import pallas as pl
from jax.experimental.pallas import tpu as pltpu, tpu_sc as plsc

N_NODE = 10000
NPAD = 10240
T_HIS = 12
ST = 32
NPASS = 3
W = 4 * ST
E_TOT = 160000
NC, NS = 2, 16
HALF = NPAD // NC
ACC_ROWS = HALF + 8
CHUNK = 128
NCHUNK = 80
EPT = CHUNK * NCHUNK
E_PAD = EPT * NS
RPT = HALF // NS
BN = 512


def _stage1_call(xT, w1fl, b1, a1T, relT, gselfT):
    cin = xT.shape[2]
    with_self = gselfT is not None
    grid = NPAD // BN

    def body(*refs):
        if with_self:
            x_ref, w_ref, b_ref, a_ref, r_ref, g_ref, m_ref, s_ref = refs
        else:
            x_ref, w_ref, b_ref, a_ref, r_ref, m_ref = refs
        x = x_ref[...]
        w = w_ref[...]
        b = b_ref[...]
        aT = a_ref[...]
        rT = r_ref[...]
        z = jnp.zeros((BN, ST), jnp.float32)
        ms, ss = [], []
        for t in range(10):
            xw = jnp.concatenate([x[:, t, :], x[:, t + 1, :], x[:, t + 2, :]],
                                 axis=-1)
            y = jnp.dot(xw, w, preferred_element_type=jnp.float32) + b
            p, q = y[:, :ST], y[:, ST:]
            xa = jnp.dot(x[:, t + 2, :], aT, preferred_element_type=jnp.float32)
            h = (p + xa) * jax.nn.sigmoid(q)
            ms.append(jnp.dot(h, rT, preferred_element_type=jnp.float32))
            if with_self:
                ss.append(jnp.dot(h, g_ref[...],
                                  preferred_element_type=jnp.float32))
        ms += [z, z]
        ss += [z, z]
        for pp in range(NPASS):
            m_ref[:, pp, :] = jnp.concatenate(ms[4 * pp:4 * pp + 4], axis=-1)
            if with_self:
                s_ref[pp, :, :] = jnp.concatenate(ss[4 * pp:4 * pp + 4],
                                                  axis=-1)

    full = lambda *s: pl.BlockSpec(s, lambda i: (0,) * len(s))
    in_specs = [
        pl.BlockSpec((BN, T_HIS, cin), lambda i: (i, 0, 0)),
        full(3 * cin, 2 * ST),
        full(1, 2 * ST),
        full(cin, ST),
        full(ST, ST),
    ]
    out_specs = [pl.BlockSpec((BN, NPASS, W), lambda i: (i, 0, 0))]
    out_shape = [jax.ShapeDtypeStruct((NPAD, NPASS, W), jnp.float32)]
    args = [xT, w1fl, b1, a1T, relT]
    if with_self:
        in_specs.append(full(ST, ST))
        out_specs.append(pl.BlockSpec((NPASS, BN, W), lambda i: (0, i, 0)))
        out_shape.append(jax.ShapeDtypeStruct((NPASS, NPAD, W), jnp.float32))
        args.append(gselfT)
    out = pl.pallas_call(
        body, grid=(grid,), in_specs=in_specs, out_specs=out_specs,
        out_shape=out_shape)(*args)
    return out if with_self else (out[0], None)


def _sc_aggregate(mm3, ma3, ss, src_sr, dst_sr, src_rr, dst_rr):
    mesh = plsc.VectorSubcoreMesh(core_axis_name="c", subcore_axis_name="s",
                                  num_cores=NC, num_subcores=NS)

    @functools.partial(
        pl.kernel,
        out_type=jax.ShapeDtypeStruct((NPASS, NPAD, W), jnp.float32),
        mesh=mesh,
        scratch_types=[
            pltpu.VMEM((2, NCHUNK + 1, CHUNK), jnp.int32),
            pltpu.VMEM((2, NCHUNK, CHUNK), jnp.int32),
            pltpu.VMEM((CHUNK, W), jnp.float32),
            pltpu.VMEM((CHUNK, W), jnp.float32),
            pltpu.VMEM_SHARED((ACC_ROWS, W), jnp.float32),
            pltpu.SemaphoreType.DMA,
            pltpu.SemaphoreType.DMA,
            pltpu.SemaphoreType.DMA,
            pltpu.SemaphoreType.DMA,
        ],
    )
    def k(mm_hbm, ma_hbm, ss_hbm, ssr_hbm, dsr_hbm, srr_hbm, drr_hbm, out_hbm,
          gi, dloc, rows0, rows1, acc, semg0, semg1, sems0, sems1):
        cid = lax.axis_index("c")
        sid = lax.axis_index("s")
        base = sid * RPT
        zed = jnp.zeros((16,), jnp.int32)

        for r, (s_hbm, d_hbm) in enumerate(((ssr_hbm, dsr_hbm),
                                            (srr_hbm, drr_hbm))):
            pltpu.sync_copy(s_hbm.at[sid], gi.at[r, pl.ds(0, NCHUNK)])
            pltpu.sync_copy(d_hbm.at[sid], dloc.at[r])

            def prep(i, _, r=r):
                for g in range(CHUNK // 16):
                    sl = pl.ds(g * 16, 16)
                    gi[r, i, sl] = gi[r, i, sl] * 3
                    dl = dloc[r, i, sl] - cid * HALF
                    ok = (dl >= 0) & (dl < HALF)
                    dloc[r, i, sl] = jnp.where(ok, dl, HALF)
                return 0

            lax.fori_loop(0, NCHUNK, prep, 0)
            for g in range(CHUNK // 16):
                gi[r, NCHUNK, pl.ds(g * 16, 16)] = zed

        for p in range(NPASS):
            if p > 0:
                def bump(i, _):
                    for r in range(2):
                        for g in range(CHUNK // 16):
                            sl = pl.ds(g * 16, 16)
                            gi[r, i, sl] = gi[r, i, sl] + 1
                    return 0

                lax.fori_loop(0, NCHUNK, bump, 0)
            pltpu.sync_copy(ss_hbm.at[p, pl.ds(cid * HALF + base, RPT)],
                            acc.at[pl.ds(base, RPT)])
            plsc.subcore_barrier()
            for r, tab in enumerate((mm_hbm, ma_hbm)):
                pltpu.async_copy(tab.at[gi.at[r, 0]], rows0, semg0)

                def body2(j, _, r=r, tab=tab):
                    i0 = 2 * j
                    pltpu.async_copy(tab.at[gi.at[r, i0 + 1]], rows1, semg1)
                    pltpu.make_async_copy(tab.at[gi.at[r, i0]], rows0,
                                          semg0).wait()
                    pltpu.async_copy(rows0, acc.at[dloc.at[r, i0]], sems0,
                                     add=True).wait()
                    pltpu.async_copy(tab.at[gi.at[r, i0 + 2]], rows0, semg0)
                    pltpu.make_async_copy(tab.at[gi.at[r, i0 + 1]], rows1,
                                          semg1).wait()
                    pltpu.async_copy(rows1, acc.at[dloc.at[r, i0 + 1]], sems1,
                                     add=True).wait()
                    return 0

                lax.fori_loop(0, NCHUNK // 2, body2, 0)
                pltpu.make_async_copy(tab.at[gi.at[r, NCHUNK]], rows0,
                                      semg0).wait()
            plsc.subcore_barrier()
            pltpu.sync_copy(acc.at[pl.ds(base, RPT)],
                            out_hbm.at[p, pl.ds(cid * HALF + base, RPT)])
            if p + 1 < NPASS:
                plsc.subcore_barrier()

    return k(mm3, ma3, ss, src_sr, dst_sr, src_rr, dst_rr)


def _stage2_call(agg3, w2fl, b2, a2T, lng, lnb, twT, tb, taT, lng2, lnb2,
                 f1T, f1b, f2T, f2b):
    grid = NPAD // BN

    def body(ag_ref, w2_ref, b2_ref, a2_ref, g_ref, bt_ref, tw_ref, tb_ref,
             ta_ref, g2_ref, bt2_ref, f1_ref, f1b_ref, f2_ref, f2b_ref, o_ref):
        ags = []
        for pp in range(NPASS):
            blkp = jnp.maximum(ag_ref[pp], 0.0)
            for j in range(4):
                ags.append(blkp[:, j * ST:(j + 1) * ST])
        w2 = w2_ref[...]
        hs = []
        for t in range(8):
            xw = jnp.concatenate([ags[t], ags[t + 1], ags[t + 2]], axis=-1)
            y = jnp.dot(xw, w2, preferred_element_type=jnp.float32) + b2_ref[...]
            p, q = y[:, :ST], y[:, ST:]
            xa = jnp.dot(ags[t + 2], a2_ref[...],
                         preferred_element_type=jnp.float32)
            h = (p + xa) * jax.nn.sigmoid(q)
            mu = jnp.mean(h, axis=-1, keepdims=True)
            var = jnp.mean((h - mu) ** 2, axis=-1, keepdims=True)
            hs.append((h - mu) * lax.rsqrt(var + 1e-5) * g_ref[...]
                      + bt_ref[...])
        acc = tb_ref[...] + jnp.zeros((BN, 256), jnp.float32)
        for t in range(8):
            acc = acc + jnp.dot(hs[t], tw_ref[t],
                                preferred_element_type=jnp.float32)
        p, q = acc[:, :128], acc[:, 128:]
        xa = jnp.dot(hs[7], ta_ref[...], preferred_element_type=jnp.float32)
        z = (p + xa) * jax.nn.sigmoid(q)
        mu = jnp.mean(z, axis=-1, keepdims=True)
        var = jnp.mean((z - mu) ** 2, axis=-1, keepdims=True)
        z = (z - mu) * lax.rsqrt(var + 1e-5) * g2_ref[...] + bt2_ref[...]
        z = jnp.maximum(
            jnp.dot(z, f1_ref[...], preferred_element_type=jnp.float32)
            + f1b_ref[...], 0.0)
        o_ref[...] = (jnp.dot(z, f2_ref[...], preferred_element_type=jnp.float32)
                      + f2b_ref[...])

    full = lambda *s: pl.BlockSpec(s, lambda i: (0,) * len(s))
    return pl.pallas_call(
        body,
        grid=(grid,),
        in_specs=[
            pl.BlockSpec((NPASS, BN, W), lambda i: (0, i, 0)),
            full(3 * ST, 2 * ST), full(1, 2 * ST), full(ST, ST),
            full(1, ST), full(1, ST),
            full(8, ST, 256), full(1, 256), full(ST, 128),
            full(1, 128), full(1, 128),
            full(128, 128), full(1, 128), full(128, 8), full(1, 8),
        ],
        out_specs=pl.BlockSpec((BN, 8), lambda i: (i, 0)),
        out_shape=jax.ShapeDtypeStruct((NPAD, 8), jnp.float32),
    )(agg3, w2fl, b2, a2T, lng, lnb, twT, tb, taT, lng2, lnb2, f1T, f1b,
      f2T, f2b)


def _node_major(x, npad):
    xt = jnp.transpose(x[0], (2, 1, 0))
    return jnp.pad(xt, ((0, npad - xt.shape[0]), (0, 0), (0, 0)))


def _conv_flat(w):
    return jnp.transpose(w[..., 0], (2, 1, 0)).reshape(-1, w.shape[0])


def _pad_edges(ei):
    src = jnp.pad(ei[0], (0, E_PAD - E_TOT), constant_values=0)
    dst = jnp.pad(ei[1], (0, E_PAD - E_TOT), constant_values=1 << 20)
    return (src.reshape(NS, NCHUNK, CHUNK), dst.reshape(NS, NCHUNK, CHUNK))


def kernel(x_room, x_sensor, edge_index_sr, edge_index_rr, params):
    blk = params["blocks"][0]
    po = params["out"]
    br, bs = blk["room"], blk["sensor"]

    xr = _node_major(x_room, NPAD)
    xs = _node_major(x_sensor, NPAD)

    m_adj, s_self = _stage1_call(
        xr, _conv_flat(br["t1_w"]), br["t1_b"].reshape(1, -1),
        br["t1_align"].T, blk["rel"]["adjacent"].T, br["g_self"].T)
    m_meas, _ = _stage1_call(
        xs, _conv_flat(bs["t1_w"]), bs["t1_b"].reshape(1, -1),
        bs["t1_align"].T, blk["rel"]["measures"].T, None)

    ssr, dsr = _pad_edges(edge_index_sr)
    srr, drr = _pad_edges(edge_index_rr)
    agg3 = _sc_aggregate(
        m_meas.reshape(NPASS * NPAD, W), m_adj.reshape(NPASS * NPAD, W),
        s_self, ssr, dsr, srr, drr)

    f2T = jnp.zeros((128, 8), jnp.float32).at[:, :3].set(po["fc2_w"].T)
    f2b = jnp.zeros((1, 8), jnp.float32).at[:, :3].set(po["fc2_b"][None, :])
    out = _stage2_call(
        agg3,
        _conv_flat(br["t2_w"]), br["t2_b"].reshape(1, -1), br["t2_align"].T,
        br["ln_g"].reshape(1, -1), br["ln_b"].reshape(1, -1),
        jnp.transpose(po["t_w"][..., 0], (2, 1, 0)),
        po["t_b"].reshape(1, -1), po["t_align"].T,
        po["ln_g"].reshape(1, -1), po["ln_b"].reshape(1, -1),
        po["fc1_w"].T, po["fc1_b"].reshape(1, -1), f2T, f2b)

    y = out[:N_NODE, :3]
    return jnp.transpose(y, (1, 0))[None, :, :]
